# Initial kernel scaffold; baseline (speedup 1.0000x reference)
import jax
import jax.numpy as jnp
from jax.experimental import pallas as pl


def kernel(Xv_in, v, e, W1, b1, W2, b2, gamma, beta):
    raise NotImplementedError("write your pallas kernel here")


# speedup vs baseline: 6.0903x; 6.0903x over previous
import jax
import jax.numpy as jnp
from jax import lax
from jax.experimental import pallas as pl
from jax.experimental.pallas import tpu as pltpu
from jax.experimental.pallas import tpu_sc as plsc

N = 10000
D = 128
NNZ = 320000
M = 20000
M2 = M // 2

NC = 2
NS = 16
B = 80

_mesh = plsc.VectorSubcoreMesh(core_axis_name="c", subcore_axis_name="s")


def _k1_body(xv_ref, w1_ref, b1_ref, w2b_ref, ea1_ref, sv_ref):
    x = xv_ref[...]
    a1 = jnp.dot(x, w1_ref[...], preferred_element_type=jnp.float32) + b1_ref[0, 0]
    ea1_ref[...] = jnp.exp(a1 - jnp.max(a1))
    sv_ref[...] = jnp.dot(x, w2b_ref[...], preferred_element_type=jnp.float32)


def _k1(Xv_in, W1, b1, W2b):
    return pl.pallas_call(
        _k1_body,
        out_shape=[
            jax.ShapeDtypeStruct((N, 1), jnp.float32),
            jax.ShapeDtypeStruct((N, 1), jnp.float32),
        ],
    )(Xv_in, W1, b1.reshape(1, 1), W2b)


def _zero_bufs(scb, gb):
    zv = jnp.zeros((16,), jnp.float32)

    def zrow(r, carry):
        for k in range(D // 16):
            scb[r, pl.ds(k * 16, 16)] = zv
        return carry

    lax.fori_loop(0, B, zrow, 0)

    def zg(t, carry):
        gb[pl.ds(t * 16, 16)] = zv
        return carry

    lax.fori_loop(0, B // 16, zg, 0)


def _k2_body(xv_hbm, v_hbm, e_hbm, ea1_hbm,
             xe_hbm, ses1_hbm,
             vb, eb, ib, db, rowb, scb, gb, accS, sesS, sem):
    c = lax.axis_index("c")
    s = lax.axis_index("s")
    _zero_bufs(scb, gb)
    lanes = lax.iota(jnp.int32, 16)

    def fill_ib(base):
        def fill(t, carry):
            ib[pl.ds(t * 16, 16)] = lanes + (base + t * 16)
            return carry

        lax.fori_loop(0, B // 16, fill, 0)

    def zacc(i, carry):
        idx = s + i * NS

        @pl.when(idx < M2 // B)
        def _():
            fill_ib(idx * B)
            pltpu.sync_copy(scb, accS.at[ib])

        return carry

    lax.fori_loop(0, (M2 // B + NS - 1) // NS, zacc, 0)

    @pl.when(s == 10)
    def _():
        fill_ib(M2)
        pltpu.sync_copy(scb, accS.at[ib])

    @pl.when(c == 0)
    def _():
        def zses(i, carry):
            idx = s + i * NS

            @pl.when(idx < M // B)
            def _():
                fill_ib(idx * B)
                pltpu.sync_copy(gb, sesS.at[ib])

            return carry

        lax.fori_loop(0, (M // B + NS - 1) // NS, zses, 0)

    plsc.subcore_barrier()

    chunk = NNZ // NS
    base = s * chunk
    cm = c * M2

    def batch(j, carry):
        off = base + j * B
        pltpu.sync_copy(v_hbm.at[pl.ds(off, B)], vb)
        pltpu.sync_copy(e_hbm.at[pl.ds(off, B)], eb)
        pltpu.async_copy(xv_hbm.at[vb], rowb, sem).wait()
        pltpu.sync_copy(ea1_hbm.at[vb], gb)

        def grp(t, carry2):
            sl = pl.ds(t * 16, 16)
            g = gb[sl]
            ee = eb[sl] - cm
            inr = jnp.logical_and(ee >= 0, ee < M2)
            db[sl] = jnp.where(inr, ee, M2 + t * 16 + lanes)
            for rr in range(16):
                gs = g[rr]
                r = t * 16 + rr
                for k in range(D // 16):
                    scb[r, pl.ds(k * 16, 16)] = (
                        rowb[r, pl.ds(k * 16, 16)] * gs)
            return carry2

        lax.fori_loop(0, B // 16, grp, 0)

        pltpu.sync_copy(scb, accS.at[db], add=True)

        @pl.when(c == 0)
        def _():
            pltpu.sync_copy(gb, sesS.at[eb], add=True)

        return carry

    lax.fori_loop(0, NNZ // NS // B, batch, 0)
    plsc.subcore_barrier()

    def wout(i, carry):
        idx = s + i * NS

        @pl.when(idx < M2 // B)
        def _():
            fill_ib(idx * B)
            pltpu.async_copy(accS.at[ib], scb, sem).wait()
            pltpu.sync_copy(scb, xe_hbm.at[pl.ds(cm + idx * B, B)])

        return carry

    lax.fori_loop(0, (M2 // B + NS - 1) // NS, wout, 0)

    @pl.when(c == 0)
    def _():
        def wses(i, carry):
            idx = s + i * NS

            @pl.when(idx < M // B)
            def _():
                fill_ib(idx * B)
                pltpu.async_copy(sesS.at[ib], gb, sem).wait()
                pltpu.sync_copy(gb, ses1_hbm.at[pl.ds(idx * B, B)])

            return carry

        lax.fori_loop(0, (M // B + NS - 1) // NS, wses, 0)


_k2 = pl.kernel(
    _k2_body,
    out_type=[
        jax.ShapeDtypeStruct((M, D), jnp.float32),
        jax.ShapeDtypeStruct((M,), jnp.float32),
    ],
    mesh=_mesh,
    scratch_types=[
        pltpu.VMEM((B,), jnp.int32),
        pltpu.VMEM((B,), jnp.int32),
        pltpu.VMEM((B,), jnp.int32),
        pltpu.VMEM((B,), jnp.int32),
        pltpu.VMEM((B, D), jnp.float32),
        pltpu.VMEM((B, D), jnp.float32),
        pltpu.VMEM((B,), jnp.float32),
        pltpu.VMEM_SHARED((M2 + B, D), jnp.float32),
        pltpu.VMEM_SHARED((M,), jnp.float32),
        pltpu.SemaphoreType.DMA,
    ],
)


def _k3_body(xen_ref, ses1_ref, g_ref, b_ref, w2a_ref, b2_ref,
             xe_ref, e1_ref):
    x = xen_ref[...] / (ses1_ref[...] + 1e-16)
    mu = jnp.mean(x, axis=-1, keepdims=True)
    xc = x - mu
    var = jnp.mean(xc * xc, axis=-1, keepdims=True)
    xn = xc * lax.rsqrt(var + 1e-5) * g_ref[...] + b_ref[...]
    xe_ref[...] = xn
    se = jnp.dot(xn, w2a_ref[...], preferred_element_type=jnp.float32) + b2_ref[0, 0]
    e1_ref[...] = jnp.exp(se)


_BM = 2000


def _k3(Xe_num, ses1, gamma, beta, W2a, b2):
    blk = lambda cdim: pl.BlockSpec((_BM, cdim), lambda i: (i, 0))
    fix = lambda r, cdim: pl.BlockSpec((r, cdim), lambda i: (0, 0))
    return pl.pallas_call(
        _k3_body,
        grid=(M // _BM,),
        in_specs=[blk(D), blk(1), fix(1, D), fix(1, D), fix(D, 1), fix(1, 1)],
        out_specs=[blk(D), blk(1)],
        out_shape=[
            jax.ShapeDtypeStruct((M, D), jnp.float32),
            jax.ShapeDtypeStruct((M, 1), jnp.float32),
        ],
    )(Xe_num, ses1.reshape(M, 1), gamma.reshape(1, D), beta.reshape(1, D),
      W2a, b2.reshape(1, 1))


def _k4_body(xe_hbm, v_hbm, e_hbm, ea1_hbm, sv_hbm, e1_hbm, ses1_hbm,
             xva_hbm, xvb_hbm, s2a_hbm, s2b_hbm,
             vb, eb, ib, rowb, scb, gb, ab, svb, e1b, s1b, accS, sesS, sem):
    c = lax.axis_index("c")
    s = lax.axis_index("s")
    _zero_bufs(scb, gb)
    lanes = lax.iota(jnp.int32, 16)

    def fill_ib(base):
        def fill(t, carry):
            ib[pl.ds(t * 16, 16)] = lanes + (base + t * 16)
            return carry

        lax.fori_loop(0, B // 16, fill, 0)

    def zacc(i, carry):
        idx = s + i * NS

        @pl.when(idx < N // B)
        def _():
            fill_ib(idx * B)
            pltpu.sync_copy(scb, accS.at[ib])
            pltpu.sync_copy(gb, sesS.at[ib])

        return carry

    lax.fori_loop(0, (N // B + NS - 1) // NS, zacc, 0)

    plsc.subcore_barrier()

    chunk = NNZ // NC // NS
    base = c * (NNZ // NC) + s * chunk

    def batch(j, carry):
        off = base + j * B
        pltpu.sync_copy(v_hbm.at[pl.ds(off, B)], vb)
        pltpu.sync_copy(e_hbm.at[pl.ds(off, B)], eb)
        pltpu.async_copy(xe_hbm.at[eb], rowb, sem).wait()
        pltpu.sync_copy(ea1_hbm.at[vb], ab)
        pltpu.sync_copy(sv_hbm.at[vb], svb)
        pltpu.sync_copy(e1_hbm.at[eb], e1b)
        pltpu.sync_copy(ses1_hbm.at[eb], s1b)

        def grp(t, carry2):
            sl = pl.ds(t * 16, 16)
            w1 = ab[sl] / (s1b[sl] + 1e-16)
            g = e1b[sl] * jnp.exp(w1 * svb[sl])
            gb[sl] = g
            for rr in range(16):
                gs = g[rr]
                r = t * 16 + rr
                for k in range(D // 16):
                    scb[r, pl.ds(k * 16, 16)] = (
                        rowb[r, pl.ds(k * 16, 16)] * gs)
            return carry2

        lax.fori_loop(0, B // 16, grp, 0)

        pltpu.sync_copy(scb, accS.at[vb], add=True)
        pltpu.sync_copy(gb, sesS.at[vb], add=True)
        return carry

    lax.fori_loop(0, chunk // B, batch, 0)
    plsc.subcore_barrier()

    def wout(i, carry):
        idx = s + i * NS

        @pl.when(idx < N // B)
        def _():
            sl = pl.ds(idx * B, B)
            fill_ib(idx * B)
            pltpu.async_copy(accS.at[ib], scb, sem).wait()
            pltpu.async_copy(sesS.at[ib], gb, sem).wait()

            @pl.when(c == 0)
            def _():
                pltpu.sync_copy(scb, xva_hbm.at[sl])
                pltpu.sync_copy(gb, s2a_hbm.at[sl])

            @pl.when(c == 1)
            def _():
                pltpu.sync_copy(scb, xvb_hbm.at[sl])
                pltpu.sync_copy(gb, s2b_hbm.at[sl])

        return carry

    lax.fori_loop(0, (N // B + NS - 1) // NS, wout, 0)


_k4 = pl.kernel(
    _k4_body,
    out_type=[
        jax.ShapeDtypeStruct((N, D), jnp.float32),
        jax.ShapeDtypeStruct((N, D), jnp.float32),
        jax.ShapeDtypeStruct((N,), jnp.float32),
        jax.ShapeDtypeStruct((N,), jnp.float32),
    ],
    mesh=_mesh,
    scratch_types=[
        pltpu.VMEM((B,), jnp.int32),
        pltpu.VMEM((B,), jnp.int32),
        pltpu.VMEM((B,), jnp.int32),
        pltpu.VMEM((B, D), jnp.float32),
        pltpu.VMEM((B, D), jnp.float32),
        pltpu.VMEM((B,), jnp.float32),
        pltpu.VMEM((B,), jnp.float32),
        pltpu.VMEM((B,), jnp.float32),
        pltpu.VMEM((B,), jnp.float32),
        pltpu.VMEM((B,), jnp.float32),
        pltpu.VMEM_SHARED((N, D), jnp.float32),
        pltpu.VMEM_SHARED((N,), jnp.float32),
        pltpu.SemaphoreType.DMA,
    ],
)


def _k5_body(xva_ref, xvb_ref, s2a_ref, s2b_ref, xin_ref, out_ref):
    num = xva_ref[...] + xvb_ref[...]
    den = s2a_ref[...] + s2b_ref[...] + 1e-16
    out_ref[...] = num / den + xin_ref[...]


def _k5(XvA, XvB, s2a, s2b, Xv_in):
    return pl.pallas_call(
        _k5_body,
        out_shape=jax.ShapeDtypeStruct((N, D), jnp.float32),
    )(XvA, XvB, s2a.reshape(N, 1), s2b.reshape(N, 1), Xv_in)


@jax.jit
def kernel(Xv_in, v, e, W1, b1, W2, b2, gamma, beta):
    W2a = W2[:D]
    W2b = W2[D:]

    ea1, sv = _k1(Xv_in, W1, b1, W2b)
    ea1 = ea1.reshape(N)
    sv = sv.reshape(N)

    Xe_num, ses1 = _k2(Xv_in, v, e, ea1)
    Xe, E1 = _k3(Xe_num, ses1, gamma, beta, W2a, b2)

    XvA, XvB, s2a, s2b = _k4(Xe, v, e, ea1, sv, E1.reshape(M), ses1)
    return _k5(XvA, XvB, s2a, s2b, Xv_in)


# speedup vs baseline: 7.8569x; 1.2901x over previous
import jax
import jax.numpy as jnp
from jax import lax
from jax.experimental import pallas as pl
from jax.experimental.pallas import tpu as pltpu
from jax.experimental.pallas import tpu_sc as plsc

N = 10000
D = 128
NNZ = 320000
M = 20000
M2 = M // 2

NC = 2
NS = 16
B = 80

_mesh = plsc.VectorSubcoreMesh(core_axis_name="c", subcore_axis_name="s")


def _k1_body(xv_ref, w1_ref, b1_ref, w2b_ref, ea1_ref, sv_ref):
    x = xv_ref[...]
    a1 = jnp.dot(x, w1_ref[...], preferred_element_type=jnp.float32) + b1_ref[0, 0]
    ea1_ref[...] = jnp.exp(a1 - jnp.max(a1))
    sv_ref[...] = jnp.dot(x, w2b_ref[...], preferred_element_type=jnp.float32)


def _k1(Xv_in, W1, b1, W2b):
    return pl.pallas_call(
        _k1_body,
        out_shape=[
            jax.ShapeDtypeStruct((N, 1), jnp.float32),
            jax.ShapeDtypeStruct((N, 1), jnp.float32),
        ],
    )(Xv_in, W1, b1.reshape(1, 1), W2b)


def _kc_body(e_ref, cnt_ref):
    cnt_ref[...] = jnp.sum((e_ref[...] < M2).astype(jnp.int32)).reshape(1, 1)


def _kc(e):
    return pl.pallas_call(
        _kc_body,
        out_shape=jax.ShapeDtypeStruct((1, 1), jnp.int32),
    )(e.reshape(NNZ // 128, 128))


def _zero_bufs(scb, gb):
    zv = jnp.zeros((16,), jnp.float32)

    def zrow(r, carry):
        for k in range(D // 16):
            scb[r, pl.ds(k * 16, 16)] = zv
        return carry

    lax.fori_loop(0, B, zrow, 0)

    def zg(t, carry):
        gb[pl.ds(t * 16, 16)] = zv
        return carry

    lax.fori_loop(0, B // 16, zg, 0)


def _k2_body(xv_hbm, v_hbm, e_hbm, ea1_hbm, cnt_hbm,
             xe_hbm, ses1_hbm,
             vb, eb, ib, db, rowb, scb, gb, cnt_sm, accS, sesS, sem):
    c = lax.axis_index("c")
    s = lax.axis_index("s")
    _zero_bufs(scb, gb)
    lanes = lax.iota(jnp.int32, 16)

    def fill_ib(base):
        def fill(t, carry):
            ib[pl.ds(t * 16, 16)] = lanes + (base + t * 16)
            return carry

        lax.fori_loop(0, B // 16, fill, 0)

    def zacc(i, carry):
        idx = s + i * NS

        @pl.when(idx < M2 // B)
        def _():
            fill_ib(idx * B)
            pltpu.sync_copy(scb, accS.at[ib])
            pltpu.sync_copy(gb, sesS.at[ib])

        return carry

    lax.fori_loop(0, (M2 // B + NS - 1) // NS, zacc, 0)

    @pl.when(s == 10)
    def _():
        fill_ib(M2)
        pltpu.sync_copy(scb, accS.at[ib])
        pltpu.sync_copy(gb, sesS.at[ib])

    pltpu.sync_copy(cnt_hbm, cnt_sm)
    NBT = NNZ // B
    bsplit = cnt_sm[...][0] // B
    nb = jnp.where(c == 0, jnp.minimum(bsplit + 1, NBT), NBT - bsplit)
    b0 = jnp.where(c == 0, 0, bsplit)
    trips = jnp.maximum(0, (nb - s + NS - 1) // NS)
    cm = c * M2

    def batch(i, carry):
        j = b0 + s + i * NS
        off = j * B
        pltpu.sync_copy(v_hbm.at[pl.ds(off, B)], vb)
        pltpu.sync_copy(e_hbm.at[pl.ds(off, B)], eb)
        pltpu.async_copy(xv_hbm.at[vb], rowb, sem).wait()
        pltpu.sync_copy(ea1_hbm.at[vb], gb)

        def grp(t, carry2):
            sl = pl.ds(t * 16, 16)
            g = gb[sl]
            ee = eb[sl] - cm
            inr = jnp.logical_and(ee >= 0, ee < M2)
            db[sl] = jnp.where(inr, ee, M2 + t * 16 + lanes)
            for rr in range(16):
                gs = g[rr]
                r = t * 16 + rr
                for k in range(D // 16):
                    scb[r, pl.ds(k * 16, 16)] = (
                        rowb[r, pl.ds(k * 16, 16)] * gs)
            return carry2

        lax.fori_loop(0, B // 16, grp, 0)

        pltpu.sync_copy(scb, accS.at[db], add=True)
        pltpu.sync_copy(gb, sesS.at[db], add=True)
        return carry

    lax.fori_loop(0, trips, batch, 0)
    plsc.subcore_barrier()

    def wout(i, carry):
        idx = s + i * NS

        @pl.when(idx < M2 // B)
        def _():
            fill_ib(idx * B)
            pltpu.async_copy(accS.at[ib], scb, sem).wait()
            pltpu.sync_copy(scb, xe_hbm.at[pl.ds(cm + idx * B, B)])
            pltpu.async_copy(sesS.at[ib], gb, sem).wait()
            pltpu.sync_copy(gb, ses1_hbm.at[pl.ds(cm + idx * B, B)])

        return carry

    lax.fori_loop(0, (M2 // B + NS - 1) // NS, wout, 0)


_k2 = pl.kernel(
    _k2_body,
    out_type=[
        jax.ShapeDtypeStruct((M, D), jnp.float32),
        jax.ShapeDtypeStruct((M,), jnp.float32),
    ],
    mesh=_mesh,
    scratch_types=[
        pltpu.VMEM((B,), jnp.int32),
        pltpu.VMEM((B,), jnp.int32),
        pltpu.VMEM((B,), jnp.int32),
        pltpu.VMEM((B,), jnp.int32),
        pltpu.VMEM((B, D), jnp.float32),
        pltpu.VMEM((B, D), jnp.float32),
        pltpu.VMEM((B,), jnp.float32),
        pltpu.VMEM((16,), jnp.int32),
        pltpu.VMEM_SHARED((M2 + B, D), jnp.float32),
        pltpu.VMEM_SHARED((M2 + B,), jnp.float32),
        pltpu.SemaphoreType.DMA,
    ],
)


def _k3_body(xen_ref, ses1_ref, g_ref, b_ref, w2a_ref, b2_ref,
             xe_ref, e1_ref):
    x = xen_ref[...] / (ses1_ref[...] + 1e-16)
    mu = jnp.mean(x, axis=-1, keepdims=True)
    xc = x - mu
    var = jnp.mean(xc * xc, axis=-1, keepdims=True)
    xn = xc * lax.rsqrt(var + 1e-5) * g_ref[...] + b_ref[...]
    xe_ref[...] = xn
    se = jnp.dot(xn, w2a_ref[...], preferred_element_type=jnp.float32) + b2_ref[0, 0]
    e1_ref[...] = jnp.exp(se)


_BM = 2000


def _k3(Xe_num, ses1, gamma, beta, W2a, b2):
    blk = lambda cdim: pl.BlockSpec((_BM, cdim), lambda i: (i, 0))
    fix = lambda r, cdim: pl.BlockSpec((r, cdim), lambda i: (0, 0))
    return pl.pallas_call(
        _k3_body,
        grid=(M // _BM,),
        in_specs=[blk(D), blk(1), fix(1, D), fix(1, D), fix(D, 1), fix(1, 1)],
        out_specs=[blk(D), blk(1)],
        out_shape=[
            jax.ShapeDtypeStruct((M, D), jnp.float32),
            jax.ShapeDtypeStruct((M, 1), jnp.float32),
        ],
    )(Xe_num, ses1.reshape(M, 1), gamma.reshape(1, D), beta.reshape(1, D),
      W2a, b2.reshape(1, 1))


def _k4_body(xe_hbm, v_hbm, e_hbm, ea1_hbm, sv_hbm, e1_hbm, ses1_hbm,
             xva_hbm, xvb_hbm, s2a_hbm, s2b_hbm,
             vb, eb, ib, rowb, scb, gb, ab, svb, e1b, s1b, accS, sesS, sem):
    c = lax.axis_index("c")
    s = lax.axis_index("s")
    _zero_bufs(scb, gb)
    lanes = lax.iota(jnp.int32, 16)

    def fill_ib(base):
        def fill(t, carry):
            ib[pl.ds(t * 16, 16)] = lanes + (base + t * 16)
            return carry

        lax.fori_loop(0, B // 16, fill, 0)

    def zacc(i, carry):
        idx = s + i * NS

        @pl.when(idx < N // B)
        def _():
            fill_ib(idx * B)
            pltpu.sync_copy(scb, accS.at[ib])
            pltpu.sync_copy(gb, sesS.at[ib])

        return carry

    lax.fori_loop(0, (N // B + NS - 1) // NS, zacc, 0)

    plsc.subcore_barrier()

    chunk = NNZ // NC // NS
    base = c * (NNZ // NC) + s * chunk

    def batch(j, carry):
        off = base + j * B
        pltpu.sync_copy(v_hbm.at[pl.ds(off, B)], vb)
        pltpu.sync_copy(e_hbm.at[pl.ds(off, B)], eb)
        pltpu.async_copy(xe_hbm.at[eb], rowb, sem).wait()
        pltpu.sync_copy(ea1_hbm.at[vb], ab)
        pltpu.sync_copy(sv_hbm.at[vb], svb)
        pltpu.sync_copy(e1_hbm.at[eb], e1b)
        pltpu.sync_copy(ses1_hbm.at[eb], s1b)

        def grp(t, carry2):
            sl = pl.ds(t * 16, 16)
            w1 = ab[sl] / (s1b[sl] + 1e-16)
            g = e1b[sl] * jnp.exp(w1 * svb[sl])
            gb[sl] = g
            for rr in range(16):
                gs = g[rr]
                r = t * 16 + rr
                for k in range(D // 16):
                    scb[r, pl.ds(k * 16, 16)] = (
                        rowb[r, pl.ds(k * 16, 16)] * gs)
            return carry2

        lax.fori_loop(0, B // 16, grp, 0)

        pltpu.sync_copy(scb, accS.at[vb], add=True)
        pltpu.sync_copy(gb, sesS.at[vb], add=True)
        return carry

    lax.fori_loop(0, chunk // B, batch, 0)
    plsc.subcore_barrier()

    def wout(i, carry):
        idx = s + i * NS

        @pl.when(idx < N // B)
        def _():
            sl = pl.ds(idx * B, B)
            fill_ib(idx * B)
            pltpu.async_copy(accS.at[ib], scb, sem).wait()
            pltpu.async_copy(sesS.at[ib], gb, sem).wait()

            @pl.when(c == 0)
            def _():
                pltpu.sync_copy(scb, xva_hbm.at[sl])
                pltpu.sync_copy(gb, s2a_hbm.at[sl])

            @pl.when(c == 1)
            def _():
                pltpu.sync_copy(scb, xvb_hbm.at[sl])
                pltpu.sync_copy(gb, s2b_hbm.at[sl])

        return carry

    lax.fori_loop(0, (N // B + NS - 1) // NS, wout, 0)


_k4 = pl.kernel(
    _k4_body,
    out_type=[
        jax.ShapeDtypeStruct((N, D), jnp.float32),
        jax.ShapeDtypeStruct((N, D), jnp.float32),
        jax.ShapeDtypeStruct((N,), jnp.float32),
        jax.ShapeDtypeStruct((N,), jnp.float32),
    ],
    mesh=_mesh,
    scratch_types=[
        pltpu.VMEM((B,), jnp.int32),
        pltpu.VMEM((B,), jnp.int32),
        pltpu.VMEM((B,), jnp.int32),
        pltpu.VMEM((B, D), jnp.float32),
        pltpu.VMEM((B, D), jnp.float32),
        pltpu.VMEM((B,), jnp.float32),
        pltpu.VMEM((B,), jnp.float32),
        pltpu.VMEM((B,), jnp.float32),
        pltpu.VMEM((B,), jnp.float32),
        pltpu.VMEM((B,), jnp.float32),
        pltpu.VMEM_SHARED((N, D), jnp.float32),
        pltpu.VMEM_SHARED((N,), jnp.float32),
        pltpu.SemaphoreType.DMA,
    ],
)


def _k5_body(xva_ref, xvb_ref, s2a_ref, s2b_ref, xin_ref, out_ref):
    num = xva_ref[...] + xvb_ref[...]
    den = s2a_ref[...] + s2b_ref[...] + 1e-16
    out_ref[...] = num / den + xin_ref[...]


def _k5(XvA, XvB, s2a, s2b, Xv_in):
    return pl.pallas_call(
        _k5_body,
        out_shape=jax.ShapeDtypeStruct((N, D), jnp.float32),
    )(XvA, XvB, s2a.reshape(N, 1), s2b.reshape(N, 1), Xv_in)


@jax.jit
def kernel(Xv_in, v, e, W1, b1, W2, b2, gamma, beta):
    W2a = W2[:D]
    W2b = W2[D:]

    ea1, sv = _k1(Xv_in, W1, b1, W2b)
    ea1 = ea1.reshape(N)
    sv = sv.reshape(N)

    cnt = _kc(e)
    cnt16 = jnp.broadcast_to(cnt.reshape(1), (16,)).astype(jnp.int32)
    Xe_num, ses1 = _k2(Xv_in, v, e, ea1, cnt16)
    Xe, E1 = _k3(Xe_num, ses1, gamma, beta, W2a, b2)

    XvA, XvB, s2a, s2b = _k4(Xe, v, e, ea1, sv, E1.reshape(M), ses1)
    return _k5(XvA, XvB, s2a, s2b, Xv_in)


# speedup vs baseline: 12.6846x; 1.6145x over previous
import jax
import jax.numpy as jnp
from jax import lax
from jax.experimental import pallas as pl
from jax.experimental.pallas import tpu as pltpu
from jax.experimental.pallas import tpu_sc as plsc

N = 10000
D = 128
NNZ = 320000
M = 20000
M2 = M // 2

NC = 2
NS = 16
B = 80

_mesh = plsc.VectorSubcoreMesh(core_axis_name="c", subcore_axis_name="s")


def _k1_body(xv_ref, w1_ref, b1_ref, w2b_ref, ea1_ref, sv_ref):
    x = xv_ref[...]
    a1 = jnp.dot(x, w1_ref[...], preferred_element_type=jnp.float32) + b1_ref[0, 0]
    ea1_ref[...] = jnp.exp(a1 - jnp.max(a1))
    sv_ref[...] = jnp.dot(x, w2b_ref[...], preferred_element_type=jnp.float32)


def _k1(Xv_in, W1, b1, W2b):
    return pl.pallas_call(
        _k1_body,
        out_shape=[
            jax.ShapeDtypeStruct((N, 1), jnp.float32),
            jax.ShapeDtypeStruct((N, 1), jnp.float32),
        ],
    )(Xv_in, W1, b1.reshape(1, 1), W2b)


def _kc_body(e_ref, cnt_ref):
    cnt_ref[...] = jnp.sum((e_ref[...] < M2).astype(jnp.int32)).reshape(1, 1)


def _kc(e):
    return pl.pallas_call(
        _kc_body,
        out_shape=jax.ShapeDtypeStruct((1, 1), jnp.int32),
    )(e.reshape(NNZ // 128, 128))


def _zero_bufs(scb, gb):
    zv = jnp.zeros((16,), jnp.float32)

    def zrow(r, carry):
        for k in range(D // 16):
            scb[r, pl.ds(k * 16, 16)] = zv
        return carry

    lax.fori_loop(0, B, zrow, 0)

    def zg(t, carry):
        gb[pl.ds(t * 16, 16)] = zv
        return carry

    lax.fori_loop(0, B // 16, zg, 0)


def _k2_body(xv_hbm, v_hbm, e_hbm, ea1_hbm, cnt_hbm,
             xe_hbm, ses1_hbm,
             vb, eb, ib, db, rowb, scb, gb, cnt_sm, accS, sesS, sem):
    c = lax.axis_index("c")
    s = lax.axis_index("s")
    _zero_bufs(scb, gb)
    lanes = lax.iota(jnp.int32, 16)

    def fill_ib(base):
        def fill(t, carry):
            ib[pl.ds(t * 16, 16)] = lanes + (base + t * 16)
            return carry

        lax.fori_loop(0, B // 16, fill, 0)

    def zacc(i, carry):
        idx = s + i * NS

        @pl.when(idx < M2 // B)
        def _():
            fill_ib(idx * B)
            pltpu.sync_copy(scb, accS.at[ib])
            pltpu.sync_copy(gb, sesS.at[ib])

        return carry

    lax.fori_loop(0, (M2 // B + NS - 1) // NS, zacc, 0)

    @pl.when(s == 10)
    def _():
        fill_ib(M2)
        pltpu.sync_copy(scb, accS.at[ib])
        pltpu.sync_copy(gb, sesS.at[ib])

    pltpu.sync_copy(cnt_hbm, cnt_sm)
    NBT = NNZ // B
    bsplit = cnt_sm[...][0] // B
    nb = jnp.where(c == 0, jnp.minimum(bsplit + 1, NBT), NBT - bsplit)
    b0 = jnp.where(c == 0, 0, bsplit)
    trips = jnp.maximum(0, (nb - s + NS - 1) // NS)
    cm = c * M2

    def batch(i, carry):
        j = b0 + s + i * NS
        off = j * B
        d1 = pltpu.async_copy(v_hbm.at[pl.ds(off, B)], vb, sem)
        d2 = pltpu.async_copy(e_hbm.at[pl.ds(off, B)], eb, sem)
        d1.wait()
        d2.wait()
        d3 = pltpu.async_copy(xv_hbm.at[vb], rowb, sem)
        d4 = pltpu.async_copy(ea1_hbm.at[vb], gb, sem)
        d3.wait()
        d4.wait()

        def grp(t, carry2):
            sl = pl.ds(t * 16, 16)
            g = gb[sl]
            ee = eb[sl] - cm
            inr = jnp.logical_and(ee >= 0, ee < M2)
            db[sl] = jnp.where(inr, ee, M2 + t * 16 + lanes)
            for rr in range(16):
                gs = g[rr]
                r = t * 16 + rr
                for k in range(D // 16):
                    scb[r, pl.ds(k * 16, 16)] = (
                        rowb[r, pl.ds(k * 16, 16)] * gs)
            return carry2

        lax.fori_loop(0, B // 16, grp, 0)

        a1 = pltpu.async_copy(scb, accS.at[db], sem, add=True)
        a2 = pltpu.async_copy(gb, sesS.at[db], sem, add=True)
        a1.wait()
        a2.wait()
        return carry

    lax.fori_loop(0, trips, batch, 0)
    plsc.subcore_barrier()

    def wout(i, carry):
        idx = s + i * NS

        @pl.when(idx < M2 // B)
        def _():
            fill_ib(idx * B)
            pltpu.async_copy(accS.at[ib], scb, sem).wait()
            pltpu.sync_copy(scb, xe_hbm.at[pl.ds(cm + idx * B, B)])
            pltpu.async_copy(sesS.at[ib], gb, sem).wait()
            pltpu.sync_copy(gb, ses1_hbm.at[pl.ds(cm + idx * B, B)])

        return carry

    lax.fori_loop(0, (M2 // B + NS - 1) // NS, wout, 0)


_k2 = pl.kernel(
    _k2_body,
    out_type=[
        jax.ShapeDtypeStruct((M, D), jnp.float32),
        jax.ShapeDtypeStruct((M,), jnp.float32),
    ],
    mesh=_mesh,
    scratch_types=[
        pltpu.VMEM((B,), jnp.int32),
        pltpu.VMEM((B,), jnp.int32),
        pltpu.VMEM((B,), jnp.int32),
        pltpu.VMEM((B,), jnp.int32),
        pltpu.VMEM((B, D), jnp.float32),
        pltpu.VMEM((B, D), jnp.float32),
        pltpu.VMEM((B,), jnp.float32),
        pltpu.VMEM((16,), jnp.int32),
        pltpu.VMEM_SHARED((M2 + B, D), jnp.float32),
        pltpu.VMEM_SHARED((M2 + B,), jnp.float32),
        pltpu.SemaphoreType.DMA,
    ],
)


def _k3_body(xen_ref, ses1_ref, g_ref, b_ref, w2a_ref, b2_ref,
             xe_ref, e1_ref):
    x = xen_ref[...] / (ses1_ref[...] + 1e-16)
    mu = jnp.mean(x, axis=-1, keepdims=True)
    xc = x - mu
    var = jnp.mean(xc * xc, axis=-1, keepdims=True)
    xn = xc * lax.rsqrt(var + 1e-5) * g_ref[...] + b_ref[...]
    xe_ref[...] = xn
    se = jnp.dot(xn, w2a_ref[...], preferred_element_type=jnp.float32) + b2_ref[0, 0]
    e1_ref[...] = jnp.exp(se)


_BM = 2000


def _k3(Xe_num, ses1, gamma, beta, W2a, b2):
    blk = lambda cdim: pl.BlockSpec((_BM, cdim), lambda i: (i, 0))
    fix = lambda r, cdim: pl.BlockSpec((r, cdim), lambda i: (0, 0))
    return pl.pallas_call(
        _k3_body,
        grid=(M // _BM,),
        in_specs=[blk(D), blk(1), fix(1, D), fix(1, D), fix(D, 1), fix(1, 1)],
        out_specs=[blk(D), blk(1)],
        out_shape=[
            jax.ShapeDtypeStruct((M, D), jnp.float32),
            jax.ShapeDtypeStruct((M, 1), jnp.float32),
        ],
    )(Xe_num, ses1.reshape(M, 1), gamma.reshape(1, D), beta.reshape(1, D),
      W2a, b2.reshape(1, 1))


def _k4_body(xe_hbm, v_hbm, e_hbm, ea1_hbm, sv_hbm, e1_hbm, ses1_hbm,
             xva_hbm, xvb_hbm, s2a_hbm, s2b_hbm,
             vb, eb, ib, rowb, scb, gb, ab, svb, e1b, s1b, accS, sesS, sem):
    c = lax.axis_index("c")
    s = lax.axis_index("s")
    _zero_bufs(scb, gb)
    lanes = lax.iota(jnp.int32, 16)

    def fill_ib(base):
        def fill(t, carry):
            ib[pl.ds(t * 16, 16)] = lanes + (base + t * 16)
            return carry

        lax.fori_loop(0, B // 16, fill, 0)

    def zacc(i, carry):
        idx = s + i * NS

        @pl.when(idx < N // B)
        def _():
            fill_ib(idx * B)
            pltpu.sync_copy(scb, accS.at[ib])
            pltpu.sync_copy(gb, sesS.at[ib])

        return carry

    lax.fori_loop(0, (N // B + NS - 1) // NS, zacc, 0)

    plsc.subcore_barrier()

    chunk = NNZ // NC // NS
    base = c * (NNZ // NC) + s * chunk

    def batch(j, carry):
        off = base + j * B
        d1 = pltpu.async_copy(v_hbm.at[pl.ds(off, B)], vb, sem)
        d2 = pltpu.async_copy(e_hbm.at[pl.ds(off, B)], eb, sem)
        d1.wait()
        d2.wait()
        g1 = pltpu.async_copy(xe_hbm.at[eb], rowb, sem)
        g2 = pltpu.async_copy(ea1_hbm.at[vb], ab, sem)
        g3 = pltpu.async_copy(sv_hbm.at[vb], svb, sem)
        g4 = pltpu.async_copy(e1_hbm.at[eb], e1b, sem)
        g5 = pltpu.async_copy(ses1_hbm.at[eb], s1b, sem)
        g1.wait()
        g2.wait()
        g3.wait()
        g4.wait()
        g5.wait()

        def grp(t, carry2):
            sl = pl.ds(t * 16, 16)
            w1 = ab[sl] / (s1b[sl] + 1e-16)
            g = e1b[sl] * jnp.exp(w1 * svb[sl])
            gb[sl] = g
            for rr in range(16):
                gs = g[rr]
                r = t * 16 + rr
                for k in range(D // 16):
                    scb[r, pl.ds(k * 16, 16)] = (
                        rowb[r, pl.ds(k * 16, 16)] * gs)
            return carry2

        lax.fori_loop(0, B // 16, grp, 0)

        a1 = pltpu.async_copy(scb, accS.at[vb], sem, add=True)
        a2 = pltpu.async_copy(gb, sesS.at[vb], sem, add=True)
        a1.wait()
        a2.wait()
        return carry

    lax.fori_loop(0, chunk // B, batch, 0)
    plsc.subcore_barrier()

    def wout(i, carry):
        idx = s + i * NS

        @pl.when(idx < N // B)
        def _():
            sl = pl.ds(idx * B, B)
            fill_ib(idx * B)
            pltpu.async_copy(accS.at[ib], scb, sem).wait()
            pltpu.async_copy(sesS.at[ib], gb, sem).wait()

            @pl.when(c == 0)
            def _():
                pltpu.sync_copy(scb, xva_hbm.at[sl])
                pltpu.sync_copy(gb, s2a_hbm.at[sl])

            @pl.when(c == 1)
            def _():
                pltpu.sync_copy(scb, xvb_hbm.at[sl])
                pltpu.sync_copy(gb, s2b_hbm.at[sl])

        return carry

    lax.fori_loop(0, (N // B + NS - 1) // NS, wout, 0)


_k4 = pl.kernel(
    _k4_body,
    out_type=[
        jax.ShapeDtypeStruct((N, D), jnp.float32),
        jax.ShapeDtypeStruct((N, D), jnp.float32),
        jax.ShapeDtypeStruct((N,), jnp.float32),
        jax.ShapeDtypeStruct((N,), jnp.float32),
    ],
    mesh=_mesh,
    scratch_types=[
        pltpu.VMEM((B,), jnp.int32),
        pltpu.VMEM((B,), jnp.int32),
        pltpu.VMEM((B,), jnp.int32),
        pltpu.VMEM((B, D), jnp.float32),
        pltpu.VMEM((B, D), jnp.float32),
        pltpu.VMEM((B,), jnp.float32),
        pltpu.VMEM((B,), jnp.float32),
        pltpu.VMEM((B,), jnp.float32),
        pltpu.VMEM((B,), jnp.float32),
        pltpu.VMEM((B,), jnp.float32),
        pltpu.VMEM_SHARED((N, D), jnp.float32),
        pltpu.VMEM_SHARED((N,), jnp.float32),
        pltpu.SemaphoreType.DMA,
    ],
)


def _k5_body(xva_ref, xvb_ref, s2a_ref, s2b_ref, xin_ref, out_ref):
    num = xva_ref[...] + xvb_ref[...]
    den = s2a_ref[...] + s2b_ref[...] + 1e-16
    out_ref[...] = num / den + xin_ref[...]


def _k5(XvA, XvB, s2a, s2b, Xv_in):
    return pl.pallas_call(
        _k5_body,
        out_shape=jax.ShapeDtypeStruct((N, D), jnp.float32),
    )(XvA, XvB, s2a.reshape(N, 1), s2b.reshape(N, 1), Xv_in)


@jax.jit
def kernel(Xv_in, v, e, W1, b1, W2, b2, gamma, beta):
    W2a = W2[:D]
    W2b = W2[D:]

    ea1, sv = _k1(Xv_in, W1, b1, W2b)
    ea1 = ea1.reshape(N)
    sv = sv.reshape(N)

    cnt = _kc(e)
    cnt16 = jnp.broadcast_to(cnt.reshape(1), (16,)).astype(jnp.int32)
    Xe_num, ses1 = _k2(Xv_in, v, e, ea1, cnt16)
    Xe, E1 = _k3(Xe_num, ses1, gamma, beta, W2a, b2)

    XvA, XvB, s2a, s2b = _k4(Xe, v, e, ea1, sv, E1.reshape(M), ses1)
    return _k5(XvA, XvB, s2a, s2b, Xv_in)


# speedup vs baseline: 13.9837x; 1.1024x over previous
import jax
import jax.numpy as jnp
from jax import lax
from jax.experimental import pallas as pl
from jax.experimental.pallas import tpu as pltpu
from jax.experimental.pallas import tpu_sc as plsc

N = 10000
D = 128
NNZ = 320000
M = 20000
M2 = M // 2

NC = 2
NS = 16
B = 80

_mesh = plsc.VectorSubcoreMesh(core_axis_name="c", subcore_axis_name="s")


def _k1_body(xv_ref, w1_ref, b1_ref, w2b_ref, ea1_ref, sv_ref):
    x = xv_ref[...]
    a1 = jnp.dot(x, w1_ref[...], preferred_element_type=jnp.float32) + b1_ref[0, 0]
    ea1_ref[...] = jnp.exp(a1 - jnp.max(a1))
    sv_ref[...] = jnp.dot(x, w2b_ref[...], preferred_element_type=jnp.float32)


def _k1(Xv_in, W1, b1, W2b):
    return pl.pallas_call(
        _k1_body,
        out_shape=[
            jax.ShapeDtypeStruct((N, 1), jnp.float32),
            jax.ShapeDtypeStruct((N, 1), jnp.float32),
        ],
    )(Xv_in, W1, b1.reshape(1, 1), W2b)


def _kc_body(e_ref, cnt_ref):
    cnt_ref[...] = jnp.sum((e_ref[...] < M2).astype(jnp.int32)).reshape(1, 1)


def _kc(e):
    return pl.pallas_call(
        _kc_body,
        out_shape=jax.ShapeDtypeStruct((1, 1), jnp.int32),
    )(e.reshape(NNZ // 128, 128))


def _zero_bufs(scb, gb):
    zv = jnp.zeros((16,), jnp.float32)

    def zrow(r, carry):
        for k in range(D // 16):
            scb[r, pl.ds(k * 16, 16)] = zv
        return carry

    lax.fori_loop(0, B, zrow, 0)

    def zg(t, carry):
        gb[pl.ds(t * 16, 16)] = zv
        return carry

    lax.fori_loop(0, B // 16, zg, 0)


def _k2_body(xv_hbm, v_hbm, e_hbm, ea1_hbm, cnt_hbm,
             xe_hbm, ses1_hbm,
             vb, eb, ib, db, rowb, scb, gb, cnt_sm, accS, sesS, sem):
    c = lax.axis_index("c")
    s = lax.axis_index("s")
    _zero_bufs(scb, gb)
    lanes = lax.iota(jnp.int32, 16)

    def fill_ib(base):
        def fill(t, carry):
            ib[pl.ds(t * 16, 16)] = lanes + (base + t * 16)
            return carry

        lax.fori_loop(0, B // 16, fill, 0)

    def zacc(i, carry):
        idx = s + i * NS

        @pl.when(idx < M2 // B)
        def _():
            fill_ib(idx * B)
            pltpu.sync_copy(scb, accS.at[ib])
            pltpu.sync_copy(gb, sesS.at[ib])

        return carry

    lax.fori_loop(0, (M2 // B + NS - 1) // NS, zacc, 0)

    @pl.when(s == 10)
    def _():
        fill_ib(M2)
        pltpu.sync_copy(scb, accS.at[ib])
        pltpu.sync_copy(gb, sesS.at[ib])

    pltpu.sync_copy(cnt_hbm, cnt_sm)
    NBT = NNZ // B
    bsplit = cnt_sm[...][0] // B
    nb = jnp.where(c == 0, jnp.minimum(bsplit + 1, NBT), NBT - bsplit)
    b0 = jnp.where(c == 0, 0, bsplit)
    trips = jnp.maximum(0, (nb - s + NS - 1) // NS)
    cm = c * M2

    def batch(i, carry):
        j = b0 + s + i * NS
        off = j * B
        d1 = pltpu.async_copy(v_hbm.at[pl.ds(off, B)], vb, sem)
        d2 = pltpu.async_copy(e_hbm.at[pl.ds(off, B)], eb, sem)
        d1.wait()
        d2.wait()
        d3 = pltpu.async_copy(xv_hbm.at[vb], rowb, sem)
        d4 = pltpu.async_copy(ea1_hbm.at[vb], gb, sem)
        d3.wait()
        d4.wait()

        def grp(t, carry2):
            sl = pl.ds(t * 16, 16)
            g = gb[sl]
            ee = eb[sl] - cm
            inr = jnp.logical_and(ee >= 0, ee < M2)
            db[sl] = jnp.where(inr, ee, M2 + t * 16 + lanes)
            for rr in range(16):
                gs = g[rr]
                r = t * 16 + rr
                for k in range(D // 16):
                    scb[r, pl.ds(k * 16, 16)] = (
                        rowb[r, pl.ds(k * 16, 16)] * gs)
            return carry2

        lax.fori_loop(0, B // 16, grp, 0)

        a1 = pltpu.async_copy(scb, accS.at[db], sem, add=True)
        a2 = pltpu.async_copy(gb, sesS.at[db], sem, add=True)
        a1.wait()
        a2.wait()
        return carry

    lax.fori_loop(0, trips, batch, 0)
    plsc.subcore_barrier()

    def wout(i, carry):
        idx = s + i * NS

        @pl.when(idx < M2 // B)
        def _():
            fill_ib(idx * B)
            pltpu.async_copy(accS.at[ib], scb, sem).wait()
            pltpu.sync_copy(scb, xe_hbm.at[pl.ds(cm + idx * B, B)])
            pltpu.async_copy(sesS.at[ib], gb, sem).wait()
            pltpu.sync_copy(gb, ses1_hbm.at[pl.ds(cm + idx * B, B)])

        return carry

    lax.fori_loop(0, (M2 // B + NS - 1) // NS, wout, 0)


_k2 = pl.kernel(
    _k2_body,
    out_type=[
        jax.ShapeDtypeStruct((M, D), jnp.float32),
        jax.ShapeDtypeStruct((M,), jnp.float32),
    ],
    mesh=_mesh,
    scratch_types=[
        pltpu.VMEM((B,), jnp.int32),
        pltpu.VMEM((B,), jnp.int32),
        pltpu.VMEM((B,), jnp.int32),
        pltpu.VMEM((B,), jnp.int32),
        pltpu.VMEM((B, D), jnp.float32),
        pltpu.VMEM((B, D), jnp.float32),
        pltpu.VMEM((B,), jnp.float32),
        pltpu.VMEM((16,), jnp.int32),
        pltpu.VMEM_SHARED((M2 + B, D), jnp.float32),
        pltpu.VMEM_SHARED((M2 + B,), jnp.float32),
        pltpu.SemaphoreType.DMA,
    ],
)


def _k3_body(xen_ref, ses1_ref, g_ref, b_ref, w2a_ref, b2_ref,
             xe_ref, e1_ref):
    x = xen_ref[...] / (ses1_ref[...] + 1e-16)
    mu = jnp.mean(x, axis=-1, keepdims=True)
    xc = x - mu
    var = jnp.mean(xc * xc, axis=-1, keepdims=True)
    xn = xc * lax.rsqrt(var + 1e-5) * g_ref[...] + b_ref[...]
    xe_ref[...] = xn
    se = jnp.dot(xn, w2a_ref[...], preferred_element_type=jnp.float32) + b2_ref[0, 0]
    e1_ref[...] = jnp.exp(se)


_BM = 2000


def _k3(Xe_num, ses1, gamma, beta, W2a, b2):
    blk = lambda cdim: pl.BlockSpec((_BM, cdim), lambda i: (i, 0))
    fix = lambda r, cdim: pl.BlockSpec((r, cdim), lambda i: (0, 0))
    return pl.pallas_call(
        _k3_body,
        grid=(M // _BM,),
        in_specs=[blk(D), blk(1), fix(1, D), fix(1, D), fix(D, 1), fix(1, 1)],
        out_specs=[blk(D), blk(1)],
        out_shape=[
            jax.ShapeDtypeStruct((M, D), jnp.float32),
            jax.ShapeDtypeStruct((M, 1), jnp.float32),
        ],
    )(Xe_num, ses1.reshape(M, 1), gamma.reshape(1, D), beta.reshape(1, D),
      W2a, b2.reshape(1, 1))


def _k4_body(xe_hbm, v_hbm, e_hbm, ea1_hbm, sv_hbm, e1_hbm, ses1_hbm,
             xva_hbm, xvb_hbm, s2a_hbm, s2b_hbm,
             vb, eb, ib, db, rowb, scb, gb, ab, svb, e1b, s1b,
             vb2, eb2, db2, rowb2, scb2, gb2, ab2, svb2, e1b2, s1b2,
             accS, sesS, sem, semv, semg, sema):
    c = lax.axis_index("c")
    s = lax.axis_index("s")
    _zero_bufs(scb, gb)
    lanes = lax.iota(jnp.int32, 16)

    def fill_ib(base):
        def fill(t, carry):
            ib[pl.ds(t * 16, 16)] = lanes + (base + t * 16)
            return carry

        lax.fori_loop(0, B // 16, fill, 0)

    def zacc(i, carry):
        idx = s + i * NS

        @pl.when(idx < N // B)
        def _():
            fill_ib(idx * B)
            pltpu.sync_copy(scb, accS.at[ib])
            pltpu.sync_copy(gb, sesS.at[ib])

        return carry

    lax.fori_loop(0, (N // B + NS - 1) // NS, zacc, 0)

    plsc.subcore_barrier()

    chunk = NNZ // NC // NS
    base = c * (NNZ // NC) + s * chunk
    NBK = chunk // B

    def ve_issue(k, vbx, ebx):
        off = base + k * B
        pltpu.async_copy(v_hbm.at[pl.ds(off, B)], vbx, semv)
        pltpu.async_copy(e_hbm.at[pl.ds(off, B)], ebx, semv)

    def ve_wait(vbx, ebx):
        pltpu.make_async_copy(v_hbm.at[pl.ds(0, B)], vbx, semv).wait()
        pltpu.make_async_copy(e_hbm.at[pl.ds(0, B)], ebx, semv).wait()

    def gather_issue(vbx, ebx, rowbx, abx, svbx, e1bx, s1bx):
        pltpu.async_copy(xe_hbm.at[ebx], rowbx, semg)
        pltpu.async_copy(ea1_hbm.at[vbx], abx, semg)
        pltpu.async_copy(sv_hbm.at[vbx], svbx, semg)
        pltpu.async_copy(e1_hbm.at[ebx], e1bx, semg)
        pltpu.async_copy(ses1_hbm.at[ebx], s1bx, semg)

    def gather_wait(vbx, ebx, rowbx, abx, svbx, e1bx, s1bx):
        pltpu.make_async_copy(xe_hbm.at[ebx], rowbx, semg).wait()
        pltpu.make_async_copy(ea1_hbm.at[vbx], abx, semg).wait()
        pltpu.make_async_copy(sv_hbm.at[vbx], svbx, semg).wait()
        pltpu.make_async_copy(e1_hbm.at[ebx], e1bx, semg).wait()
        pltpu.make_async_copy(ses1_hbm.at[ebx], s1bx, semg).wait()

    def compute(vbx, ebx, rowbx, scbx, gbx, dbx, abx, svbx, e1bx, s1bx):
        def grp(t, carry2):
            sl = pl.ds(t * 16, 16)
            dbx[sl] = vbx[sl]
            w1 = abx[sl] / (s1bx[sl] + 1e-16)
            g = e1bx[sl] * jnp.exp(w1 * svbx[sl])
            gbx[sl] = g
            for rr in range(16):
                gs = g[rr]
                r = t * 16 + rr
                for k in range(D // 16):
                    scbx[r, pl.ds(k * 16, 16)] = (
                        rowbx[r, pl.ds(k * 16, 16)] * gs)
            return carry2

        lax.fori_loop(0, B // 16, grp, 0)

    def adds_issue(scbx, gbx, dbx):
        pltpu.async_copy(scbx, accS.at[dbx], sema, add=True)
        pltpu.async_copy(gbx, sesS.at[dbx], sema, add=True)

    def adds_wait(scbx, gbx, dbx):
        pltpu.make_async_copy(scbx, accS.at[dbx], sema).wait()
        pltpu.make_async_copy(gbx, sesS.at[dbx], sema).wait()

    A = (vb, eb, rowb, ab, svb, e1b, s1b)
    Bt = (vb2, eb2, rowb2, ab2, svb2, e1b2, s1b2)

    ve_issue(0, vb, eb)

    def pair(i, carry):
        kA = 2 * i
        ve_wait(vb, eb)
        gather_issue(*A)
        ve_issue(kA + 1, vb2, eb2)
        gather_wait(*A)
        compute(vb, eb, rowb, scb, gb, db, ab, svb, e1b, s1b)
        adds_issue(scb, gb, db)
        ve_wait(vb2, eb2)
        gather_issue(*Bt)
        ve_issue(kA + 2, vb, eb)
        gather_wait(*Bt)
        compute(vb2, eb2, rowb2, scb2, gb2, db2, ab2, svb2, e1b2, s1b2)
        adds_issue(scb2, gb2, db2)
        adds_wait(scb, gb, db)
        adds_wait(scb2, gb2, db2)
        return carry

    lax.fori_loop(0, NBK // 2, pair, 0)

    ve_wait(vb, eb)
    gather_issue(*A)
    gather_wait(*A)
    compute(vb, eb, rowb, scb, gb, db, ab, svb, e1b, s1b)
    adds_issue(scb, gb, db)
    adds_wait(scb, gb, db)
    plsc.subcore_barrier()

    def wout(i, carry):
        idx = s + i * NS

        @pl.when(idx < N // B)
        def _():
            sl = pl.ds(idx * B, B)
            fill_ib(idx * B)
            pltpu.async_copy(accS.at[ib], scb, sem).wait()
            pltpu.async_copy(sesS.at[ib], gb, sem).wait()

            @pl.when(c == 0)
            def _():
                pltpu.sync_copy(scb, xva_hbm.at[sl])
                pltpu.sync_copy(gb, s2a_hbm.at[sl])

            @pl.when(c == 1)
            def _():
                pltpu.sync_copy(scb, xvb_hbm.at[sl])
                pltpu.sync_copy(gb, s2b_hbm.at[sl])

        return carry

    lax.fori_loop(0, (N // B + NS - 1) // NS, wout, 0)


_k4 = pl.kernel(
    _k4_body,
    out_type=[
        jax.ShapeDtypeStruct((N, D), jnp.float32),
        jax.ShapeDtypeStruct((N, D), jnp.float32),
        jax.ShapeDtypeStruct((N,), jnp.float32),
        jax.ShapeDtypeStruct((N,), jnp.float32),
    ],
    mesh=_mesh,
    scratch_types=[
        pltpu.VMEM((B,), jnp.int32),
        pltpu.VMEM((B,), jnp.int32),
        pltpu.VMEM((B,), jnp.int32),
        pltpu.VMEM((B,), jnp.int32),
        pltpu.VMEM((B, D), jnp.float32),
        pltpu.VMEM((B, D), jnp.float32),
        pltpu.VMEM((B,), jnp.float32),
        pltpu.VMEM((B,), jnp.float32),
        pltpu.VMEM((B,), jnp.float32),
        pltpu.VMEM((B,), jnp.float32),
        pltpu.VMEM((B,), jnp.float32),
        pltpu.VMEM((B,), jnp.int32),
        pltpu.VMEM((B,), jnp.int32),
        pltpu.VMEM((B,), jnp.int32),
        pltpu.VMEM((B, D), jnp.float32),
        pltpu.VMEM((B, D), jnp.float32),
        pltpu.VMEM((B,), jnp.float32),
        pltpu.VMEM((B,), jnp.float32),
        pltpu.VMEM((B,), jnp.float32),
        pltpu.VMEM((B,), jnp.float32),
        pltpu.VMEM((B,), jnp.float32),
        pltpu.VMEM_SHARED((N, D), jnp.float32),
        pltpu.VMEM_SHARED((N,), jnp.float32),
        pltpu.SemaphoreType.DMA,
        pltpu.SemaphoreType.DMA,
        pltpu.SemaphoreType.DMA,
        pltpu.SemaphoreType.DMA,
    ],
)


def _k5_body(xva_ref, xvb_ref, s2a_ref, s2b_ref, xin_ref, out_ref):
    num = xva_ref[...] + xvb_ref[...]
    den = s2a_ref[...] + s2b_ref[...] + 1e-16
    out_ref[...] = num / den + xin_ref[...]


def _k5(XvA, XvB, s2a, s2b, Xv_in):
    return pl.pallas_call(
        _k5_body,
        out_shape=jax.ShapeDtypeStruct((N, D), jnp.float32),
    )(XvA, XvB, s2a.reshape(N, 1), s2b.reshape(N, 1), Xv_in)


@jax.jit
def kernel(Xv_in, v, e, W1, b1, W2, b2, gamma, beta):
    W2a = W2[:D]
    W2b = W2[D:]

    ea1, sv = _k1(Xv_in, W1, b1, W2b)
    ea1 = ea1.reshape(N)
    sv = sv.reshape(N)

    cnt = _kc(e)
    cnt16 = jnp.broadcast_to(cnt.reshape(1), (16,)).astype(jnp.int32)
    Xe_num, ses1 = _k2(Xv_in, v, e, ea1, cnt16)
    Xe, E1 = _k3(Xe_num, ses1, gamma, beta, W2a, b2)

    XvA, XvB, s2a, s2b = _k4(Xe, v, e, ea1, sv, E1.reshape(M), ses1)
    return _k5(XvA, XvB, s2a, s2b, Xv_in)


# speedup vs baseline: 15.4701x; 1.1063x over previous
import jax
import jax.numpy as jnp
from jax import lax
from jax.experimental import pallas as pl
from jax.experimental.pallas import tpu as pltpu
from jax.experimental.pallas import tpu_sc as plsc

N = 10000
D = 128
NNZ = 320000
M = 20000
M2 = M // 2

NC = 2
NS = 16
B = 80

_mesh = plsc.VectorSubcoreMesh(core_axis_name="c", subcore_axis_name="s")


def _k1_body(xv_ref, w1_ref, b1_ref, w2b_ref, ea1_ref, sv_ref):
    x = xv_ref[...]
    a1 = jnp.dot(x, w1_ref[...], preferred_element_type=jnp.float32) + b1_ref[0, 0]
    ea1_ref[...] = jnp.exp(a1 - jnp.max(a1))
    sv_ref[...] = jnp.dot(x, w2b_ref[...], preferred_element_type=jnp.float32)


def _k1(Xv_in, W1, b1, W2b):
    return pl.pallas_call(
        _k1_body,
        out_shape=[
            jax.ShapeDtypeStruct((N, 1), jnp.float32),
            jax.ShapeDtypeStruct((N, 1), jnp.float32),
        ],
    )(Xv_in, W1, b1.reshape(1, 1), W2b)


def _kc_body(e_ref, cnt_ref):
    cnt_ref[...] = jnp.sum((e_ref[...] < M2).astype(jnp.int32)).reshape(1, 1)


def _kc(e):
    return pl.pallas_call(
        _kc_body,
        out_shape=jax.ShapeDtypeStruct((1, 1), jnp.int32),
    )(e.reshape(NNZ // 128, 128))


def _zero_bufs(scb, gb):
    zv = jnp.zeros((16,), jnp.float32)

    def zrow(r, carry):
        for k in range(D // 16):
            scb[r, pl.ds(k * 16, 16)] = zv
        return carry

    lax.fori_loop(0, B, zrow, 0)

    def zg(t, carry):
        gb[pl.ds(t * 16, 16)] = zv
        return carry

    lax.fori_loop(0, B // 16, zg, 0)


def _k2_body(xv_hbm, v_hbm, e_hbm, ea1_hbm, cnt_hbm,
             xe_hbm, ses1_hbm,
             vb, eb, ib, db, rowb, scb, gb,
             vb2, eb2, db2, rowb2, scb2, gb2,
             cnt_sm, accS, sesS, sem, semv, semg, sema):
    c = lax.axis_index("c")
    s = lax.axis_index("s")
    _zero_bufs(scb, gb)
    lanes = lax.iota(jnp.int32, 16)

    def fill_ib(base):
        def fill(t, carry):
            ib[pl.ds(t * 16, 16)] = lanes + (base + t * 16)
            return carry

        lax.fori_loop(0, B // 16, fill, 0)

    def zacc(i, carry):
        idx = s + i * NS

        @pl.when(idx < M2 // B)
        def _():
            fill_ib(idx * B)
            pltpu.sync_copy(scb, accS.at[ib])
            pltpu.sync_copy(gb, sesS.at[ib])

        return carry

    lax.fori_loop(0, (M2 // B + NS - 1) // NS, zacc, 0)

    @pl.when(s == 10)
    def _():
        fill_ib(M2)
        pltpu.sync_copy(scb, accS.at[ib])
        pltpu.sync_copy(gb, sesS.at[ib])

    pltpu.sync_copy(cnt_hbm, cnt_sm)
    NBT = NNZ // B
    bsplit = cnt_sm[...][0] // B
    nb = jnp.where(c == 0, jnp.minimum(bsplit + 1, NBT), NBT - bsplit)
    b0 = jnp.where(c == 0, 0, bsplit)
    trips = jnp.maximum(0, (nb - s + NS - 1) // NS)
    cm = c * M2

    def ve_issue(k, vbx, ebx):
        off = (b0 + s + k * NS) * B
        pltpu.async_copy(v_hbm.at[pl.ds(off, B)], vbx, semv)
        pltpu.async_copy(e_hbm.at[pl.ds(off, B)], ebx, semv)

    def ve_wait(vbx, ebx):
        pltpu.make_async_copy(v_hbm.at[pl.ds(0, B)], vbx, semv).wait()
        pltpu.make_async_copy(e_hbm.at[pl.ds(0, B)], ebx, semv).wait()

    def gather_issue(vbx, rowbx, gbx):
        pltpu.async_copy(xv_hbm.at[vbx], rowbx, semg)
        pltpu.async_copy(ea1_hbm.at[vbx], gbx, semg)

    def gather_wait(vbx, rowbx, gbx):
        pltpu.make_async_copy(xv_hbm.at[vbx], rowbx, semg).wait()
        pltpu.make_async_copy(ea1_hbm.at[vbx], gbx, semg).wait()

    def compute(ebx, rowbx, scbx, gbx, dbx):
        def grp(t, carry2):
            sl = pl.ds(t * 16, 16)
            g = gbx[sl]
            ee = ebx[sl] - cm
            inr = jnp.logical_and(ee >= 0, ee < M2)
            dbx[sl] = jnp.where(inr, ee, M2 + t * 16 + lanes)
            for rr in range(16):
                gs = g[rr]
                r = t * 16 + rr
                for k in range(D // 16):
                    scbx[r, pl.ds(k * 16, 16)] = (
                        rowbx[r, pl.ds(k * 16, 16)] * gs)
            return carry2

        lax.fori_loop(0, B // 16, grp, 0)

    def adds_issue(scbx, gbx, dbx):
        pltpu.async_copy(scbx, accS.at[dbx], sema, add=True)
        pltpu.async_copy(gbx, sesS.at[dbx], sema, add=True)

    def adds_wait(scbx, gbx, dbx):
        pltpu.make_async_copy(scbx, accS.at[dbx], sema).wait()
        pltpu.make_async_copy(gbx, sesS.at[dbx], sema).wait()

    @pl.when(trips > 0)
    def _():
        ve_issue(0, vb, eb)

    def pair(i, carry):
        kA = 2 * i
        ve_wait(vb, eb)
        gather_issue(vb, rowb, gb)
        ve_issue(kA + 1, vb2, eb2)
        gather_wait(vb, rowb, gb)
        compute(eb, rowb, scb, gb, db)
        adds_issue(scb, gb, db)
        ve_wait(vb2, eb2)
        gather_issue(vb2, rowb2, gb2)

        @pl.when(kA + 2 < trips)
        def _():
            ve_issue(kA + 2, vb, eb)

        gather_wait(vb2, rowb2, gb2)
        compute(eb2, rowb2, scb2, gb2, db2)
        adds_issue(scb2, gb2, db2)
        adds_wait(scb, gb, db)
        adds_wait(scb2, gb2, db2)
        return carry

    lax.fori_loop(0, trips // 2, pair, 0)

    @pl.when(trips % 2 == 1)
    def _():
        ve_wait(vb, eb)
        gather_issue(vb, rowb, gb)
        gather_wait(vb, rowb, gb)
        compute(eb, rowb, scb, gb, db)
        adds_issue(scb, gb, db)
        adds_wait(scb, gb, db)

    plsc.subcore_barrier()

    def wout(i, carry):
        idx = s + i * NS

        @pl.when(idx < M2 // B)
        def _():
            fill_ib(idx * B)
            pltpu.async_copy(accS.at[ib], scb, sem).wait()
            pltpu.sync_copy(scb, xe_hbm.at[pl.ds(cm + idx * B, B)])
            pltpu.async_copy(sesS.at[ib], gb, sem).wait()
            pltpu.sync_copy(gb, ses1_hbm.at[pl.ds(cm + idx * B, B)])

        return carry

    lax.fori_loop(0, (M2 // B + NS - 1) // NS, wout, 0)


_k2 = pl.kernel(
    _k2_body,
    out_type=[
        jax.ShapeDtypeStruct((M, D), jnp.float32),
        jax.ShapeDtypeStruct((M,), jnp.float32),
    ],
    mesh=_mesh,
    scratch_types=[
        pltpu.VMEM((B,), jnp.int32),
        pltpu.VMEM((B,), jnp.int32),
        pltpu.VMEM((B,), jnp.int32),
        pltpu.VMEM((B,), jnp.int32),
        pltpu.VMEM((B, D), jnp.float32),
        pltpu.VMEM((B, D), jnp.float32),
        pltpu.VMEM((B,), jnp.float32),
        pltpu.VMEM((B,), jnp.int32),
        pltpu.VMEM((B,), jnp.int32),
        pltpu.VMEM((B,), jnp.int32),
        pltpu.VMEM((B, D), jnp.float32),
        pltpu.VMEM((B, D), jnp.float32),
        pltpu.VMEM((B,), jnp.float32),
        pltpu.VMEM((16,), jnp.int32),
        pltpu.VMEM_SHARED((M2 + B, D), jnp.float32),
        pltpu.VMEM_SHARED((M2 + B,), jnp.float32),
        pltpu.SemaphoreType.DMA,
        pltpu.SemaphoreType.DMA,
        pltpu.SemaphoreType.DMA,
        pltpu.SemaphoreType.DMA,
    ],
)


def _k3_body(xen_ref, ses1_ref, g_ref, b_ref, w2a_ref, b2_ref,
             xe_ref, e1_ref):
    x = xen_ref[...] / (ses1_ref[...] + 1e-16)
    mu = jnp.mean(x, axis=-1, keepdims=True)
    xc = x - mu
    var = jnp.mean(xc * xc, axis=-1, keepdims=True)
    xn = xc * lax.rsqrt(var + 1e-5) * g_ref[...] + b_ref[...]
    xe_ref[...] = xn
    se = jnp.dot(xn, w2a_ref[...], preferred_element_type=jnp.float32) + b2_ref[0, 0]
    e1_ref[...] = jnp.exp(se)


_BM = 2000


def _k3(Xe_num, ses1, gamma, beta, W2a, b2):
    blk = lambda cdim: pl.BlockSpec((_BM, cdim), lambda i: (i, 0))
    fix = lambda r, cdim: pl.BlockSpec((r, cdim), lambda i: (0, 0))
    return pl.pallas_call(
        _k3_body,
        grid=(M // _BM,),
        in_specs=[blk(D), blk(1), fix(1, D), fix(1, D), fix(D, 1), fix(1, 1)],
        out_specs=[blk(D), blk(1)],
        out_shape=[
            jax.ShapeDtypeStruct((M, D), jnp.float32),
            jax.ShapeDtypeStruct((M, 1), jnp.float32),
        ],
    )(Xe_num, ses1.reshape(M, 1), gamma.reshape(1, D), beta.reshape(1, D),
      W2a, b2.reshape(1, 1))


def _k4_body(xe_hbm, v_hbm, e_hbm, ea1_hbm, sv_hbm, e1_hbm, ses1_hbm,
             xva_hbm, xvb_hbm, s2a_hbm, s2b_hbm,
             vb, eb, ib, db, rowb, scb, gb, ab, svb, e1b, s1b,
             vb2, eb2, db2, rowb2, scb2, gb2, ab2, svb2, e1b2, s1b2,
             accS, sesS, sem, semv, semg, sema):
    c = lax.axis_index("c")
    s = lax.axis_index("s")
    _zero_bufs(scb, gb)
    lanes = lax.iota(jnp.int32, 16)

    def fill_ib(base):
        def fill(t, carry):
            ib[pl.ds(t * 16, 16)] = lanes + (base + t * 16)
            return carry

        lax.fori_loop(0, B // 16, fill, 0)

    def zacc(i, carry):
        idx = s + i * NS

        @pl.when(idx < N // B)
        def _():
            fill_ib(idx * B)
            pltpu.sync_copy(scb, accS.at[ib])
            pltpu.sync_copy(gb, sesS.at[ib])

        return carry

    lax.fori_loop(0, (N // B + NS - 1) // NS, zacc, 0)

    plsc.subcore_barrier()

    chunk = NNZ // NC // NS
    base = c * (NNZ // NC) + s * chunk
    NBK = chunk // B

    def ve_issue(k, vbx, ebx):
        off = base + k * B
        pltpu.async_copy(v_hbm.at[pl.ds(off, B)], vbx, semv)
        pltpu.async_copy(e_hbm.at[pl.ds(off, B)], ebx, semv)

    def ve_wait(vbx, ebx):
        pltpu.make_async_copy(v_hbm.at[pl.ds(0, B)], vbx, semv).wait()
        pltpu.make_async_copy(e_hbm.at[pl.ds(0, B)], ebx, semv).wait()

    def gather_issue(vbx, ebx, rowbx, abx, svbx, e1bx, s1bx):
        pltpu.async_copy(xe_hbm.at[ebx], rowbx, semg)
        pltpu.async_copy(ea1_hbm.at[vbx], abx, semg)
        pltpu.async_copy(sv_hbm.at[vbx], svbx, semg)
        pltpu.async_copy(e1_hbm.at[ebx], e1bx, semg)
        pltpu.async_copy(ses1_hbm.at[ebx], s1bx, semg)

    def gather_wait(vbx, ebx, rowbx, abx, svbx, e1bx, s1bx):
        pltpu.make_async_copy(xe_hbm.at[ebx], rowbx, semg).wait()
        pltpu.make_async_copy(ea1_hbm.at[vbx], abx, semg).wait()
        pltpu.make_async_copy(sv_hbm.at[vbx], svbx, semg).wait()
        pltpu.make_async_copy(e1_hbm.at[ebx], e1bx, semg).wait()
        pltpu.make_async_copy(ses1_hbm.at[ebx], s1bx, semg).wait()

    def compute(vbx, ebx, rowbx, scbx, gbx, dbx, abx, svbx, e1bx, s1bx):
        def grp(t, carry2):
            sl = pl.ds(t * 16, 16)
            dbx[sl] = vbx[sl]
            w1 = abx[sl] / (s1bx[sl] + 1e-16)
            g = e1bx[sl] * jnp.exp(w1 * svbx[sl])
            gbx[sl] = g
            for rr in range(16):
                gs = g[rr]
                r = t * 16 + rr
                for k in range(D // 16):
                    scbx[r, pl.ds(k * 16, 16)] = (
                        rowbx[r, pl.ds(k * 16, 16)] * gs)
            return carry2

        lax.fori_loop(0, B // 16, grp, 0)

    def adds_issue(scbx, gbx, dbx):
        pltpu.async_copy(scbx, accS.at[dbx], sema, add=True)
        pltpu.async_copy(gbx, sesS.at[dbx], sema, add=True)

    def adds_wait(scbx, gbx, dbx):
        pltpu.make_async_copy(scbx, accS.at[dbx], sema).wait()
        pltpu.make_async_copy(gbx, sesS.at[dbx], sema).wait()

    A = (vb, eb, rowb, ab, svb, e1b, s1b)
    Bt = (vb2, eb2, rowb2, ab2, svb2, e1b2, s1b2)

    ve_issue(0, vb, eb)

    def pair(i, carry):
        kA = 2 * i
        ve_wait(vb, eb)
        gather_issue(*A)
        ve_issue(kA + 1, vb2, eb2)
        gather_wait(*A)
        compute(vb, eb, rowb, scb, gb, db, ab, svb, e1b, s1b)
        adds_issue(scb, gb, db)
        ve_wait(vb2, eb2)
        gather_issue(*Bt)
        ve_issue(kA + 2, vb, eb)
        gather_wait(*Bt)
        compute(vb2, eb2, rowb2, scb2, gb2, db2, ab2, svb2, e1b2, s1b2)
        adds_issue(scb2, gb2, db2)
        adds_wait(scb, gb, db)
        adds_wait(scb2, gb2, db2)
        return carry

    lax.fori_loop(0, NBK // 2, pair, 0)

    ve_wait(vb, eb)
    gather_issue(*A)
    gather_wait(*A)
    compute(vb, eb, rowb, scb, gb, db, ab, svb, e1b, s1b)
    adds_issue(scb, gb, db)
    adds_wait(scb, gb, db)
    plsc.subcore_barrier()

    def wout(i, carry):
        idx = s + i * NS

        @pl.when(idx < N // B)
        def _():
            sl = pl.ds(idx * B, B)
            fill_ib(idx * B)
            pltpu.async_copy(accS.at[ib], scb, sem).wait()
            pltpu.async_copy(sesS.at[ib], gb, sem).wait()

            @pl.when(c == 0)
            def _():
                pltpu.sync_copy(scb, xva_hbm.at[sl])
                pltpu.sync_copy(gb, s2a_hbm.at[sl])

            @pl.when(c == 1)
            def _():
                pltpu.sync_copy(scb, xvb_hbm.at[sl])
                pltpu.sync_copy(gb, s2b_hbm.at[sl])

        return carry

    lax.fori_loop(0, (N // B + NS - 1) // NS, wout, 0)


_k4 = pl.kernel(
    _k4_body,
    out_type=[
        jax.ShapeDtypeStruct((N, D), jnp.float32),
        jax.ShapeDtypeStruct((N, D), jnp.float32),
        jax.ShapeDtypeStruct((N,), jnp.float32),
        jax.ShapeDtypeStruct((N,), jnp.float32),
    ],
    mesh=_mesh,
    scratch_types=[
        pltpu.VMEM((B,), jnp.int32),
        pltpu.VMEM((B,), jnp.int32),
        pltpu.VMEM((B,), jnp.int32),
        pltpu.VMEM((B,), jnp.int32),
        pltpu.VMEM((B, D), jnp.float32),
        pltpu.VMEM((B, D), jnp.float32),
        pltpu.VMEM((B,), jnp.float32),
        pltpu.VMEM((B,), jnp.float32),
        pltpu.VMEM((B,), jnp.float32),
        pltpu.VMEM((B,), jnp.float32),
        pltpu.VMEM((B,), jnp.float32),
        pltpu.VMEM((B,), jnp.int32),
        pltpu.VMEM((B,), jnp.int32),
        pltpu.VMEM((B,), jnp.int32),
        pltpu.VMEM((B, D), jnp.float32),
        pltpu.VMEM((B, D), jnp.float32),
        pltpu.VMEM((B,), jnp.float32),
        pltpu.VMEM((B,), jnp.float32),
        pltpu.VMEM((B,), jnp.float32),
        pltpu.VMEM((B,), jnp.float32),
        pltpu.VMEM((B,), jnp.float32),
        pltpu.VMEM_SHARED((N, D), jnp.float32),
        pltpu.VMEM_SHARED((N,), jnp.float32),
        pltpu.SemaphoreType.DMA,
        pltpu.SemaphoreType.DMA,
        pltpu.SemaphoreType.DMA,
        pltpu.SemaphoreType.DMA,
    ],
)


def _k5_body(xva_ref, xvb_ref, s2a_ref, s2b_ref, xin_ref, out_ref):
    num = xva_ref[...] + xvb_ref[...]
    den = s2a_ref[...] + s2b_ref[...] + 1e-16
    out_ref[...] = num / den + xin_ref[...]


def _k5(XvA, XvB, s2a, s2b, Xv_in):
    return pl.pallas_call(
        _k5_body,
        out_shape=jax.ShapeDtypeStruct((N, D), jnp.float32),
    )(XvA, XvB, s2a.reshape(N, 1), s2b.reshape(N, 1), Xv_in)


@jax.jit
def kernel(Xv_in, v, e, W1, b1, W2, b2, gamma, beta):
    W2a = W2[:D]
    W2b = W2[D:]

    ea1, sv = _k1(Xv_in, W1, b1, W2b)
    ea1 = ea1.reshape(N)
    sv = sv.reshape(N)

    cnt = _kc(e)
    cnt16 = jnp.broadcast_to(cnt.reshape(1), (16,)).astype(jnp.int32)
    Xe_num, ses1 = _k2(Xv_in, v, e, ea1, cnt16)
    Xe, E1 = _k3(Xe_num, ses1, gamma, beta, W2a, b2)

    XvA, XvB, s2a, s2b = _k4(Xe, v, e, ea1, sv, E1.reshape(M), ses1)
    return _k5(XvA, XvB, s2a, s2b, Xv_in)
